# baseline (speedup 1.0000x reference)
import functools

import jax
import jax.numpy as jnp
from jax import lax
from jax.experimental import pallas as pl
from jax.experimental.pallas import tpu as pltpu
from jax.experimental.pallas import tpu_sc as plsc

N, C = 16384, 1000
NC, NS, L = 2, 16, 16
NW = NC * NS
RPW = N // NW
NST = RPW // L


def _sc_partial_sums(prob, target, reward):
    mesh = plsc.VectorSubcoreMesh(core_axis_name="c", subcore_axis_name="s")

    @functools.partial(
        pl.kernel,
        out_type=jax.ShapeDtypeStruct((NW * L,), jnp.float32),
        mesh=mesh,
        compiler_params=pltpu.CompilerParams(needs_layout_passes=False),
        scratch_types=[
            pltpu.VMEM((RPW,), jnp.int32),
            pltpu.VMEM((RPW,), jnp.float32),
            pltpu.VMEM((L, C), jnp.float32),
            pltpu.VMEM((L, C), jnp.float32),
            pltpu.VMEM((L,), jnp.float32),
            pltpu.SemaphoreType.DMA,
            pltpu.SemaphoreType.DMA,
        ],
    )
    def k(prob_hbm, tgt_hbm, rew_hbm, out_hbm, tgt_v, rew_v, buf0, buf1,
          acc_v, sem0, sem1):
        wid = lax.axis_index("s") * NC + lax.axis_index("c")
        base = wid * RPW
        pltpu.sync_copy(tgt_hbm.at[pl.ds(base, RPW)], tgt_v)
        pltpu.sync_copy(rew_hbm.at[pl.ds(base, RPW)], rew_v)

        bufs = (buf0, buf1)
        sems = (sem0, sem1)

        def fetch(k_):
            return pltpu.async_copy(
                prob_hbm.at[pl.ds(pl.multiple_of(base + k_ * L, 8), L), :],
                bufs[k_ % 2], sems[k_ % 2])

        lane = lax.broadcasted_iota(jnp.int32, (L,), 0)
        copies = [None, None]
        copies[0] = fetch(0)
        acc = jnp.zeros((L,), jnp.float32)
        for k_ in range(NST):
            if k_ + 1 < NST:
                copies[(k_ + 1) % 2] = fetch(k_ + 1)
            copies[k_ % 2].wait()
            t = tgt_v[pl.ds(k_ * L, L)]
            vals = plsc.load_gather(bufs[k_ % 2], [lane, t])
            acc = acc + vals * rew_v[pl.ds(k_ * L, L)]
        acc_v[...] = acc
        pltpu.sync_copy(acc_v, out_hbm.at[pl.ds(wid * L, L)])

    return k(prob, target, reward)


def kernel(prob, target, reward, device):
    partials = _sc_partial_sums(prob, target, reward)
    return -jnp.sum(partials) * (1.0 / N)
